# baseline (speedup 1.0000x reference)
import functools

import jax
import jax.numpy as jnp
from jax import lax
from jax.experimental import pallas as pl
from jax.experimental.pallas import tpu as pltpu
from jax.experimental.pallas import tpu_sc as plsc

N_HASHES = 4
Q_ATTN = 256

_NC = 2
_NS = 16
_NW = _NC * _NS

_IPG = 128
_GCHUNK = 512
_SCHUNK = 512


def _worker_id():
    return lax.axis_index("s") * _NC + lax.axis_index("c")


def _sc_gather_body(qt_hbm, kt_hbm, vt_hbm, qidx_hbm, kidx_hbm,
                    sq_hbm, sk_hbm, sv_hbm,
                    idx_q, idx_k, rq, rk, rv,
                    sg_q, sg_k, sg_v, sw_q, sw_k, sw_v):
    nrows = sq_hbm.shape[0]
    gpw = nrows // _NW
    nchunk = gpw // _GCHUNK
    npd = _GCHUNK // _IPG
    base = _worker_id() * gpw

    def chunk(c, _):
        off = base + c * _GCHUNK
        pltpu.sync_copy(qidx_hbm.at[pl.ds(off, _GCHUNK)], idx_q)
        pltpu.sync_copy(kidx_hbm.at[pl.ds(off, _GCHUNK)], idx_k)
        gq = [pltpu.async_copy(qt_hbm.at[idx_q.at[pl.ds(j * _IPG, _IPG)]],
                               rq.at[pl.ds(j * _IPG, _IPG)], sg_q)
              for j in range(npd)]
        gk = [pltpu.async_copy(kt_hbm.at[idx_k.at[pl.ds(j * _IPG, _IPG)]],
                               rk.at[pl.ds(j * _IPG, _IPG)], sg_k)
              for j in range(npd)]
        gv = [pltpu.async_copy(vt_hbm.at[idx_k.at[pl.ds(j * _IPG, _IPG)]],
                               rv.at[pl.ds(j * _IPG, _IPG)], sg_v)
              for j in range(npd)]
        for h in gq:
            h.wait()
        wq = pltpu.async_copy(rq, sq_hbm.at[pl.ds(off, _GCHUNK)], sw_q)
        for h in gk:
            h.wait()
        wk = pltpu.async_copy(rk, sk_hbm.at[pl.ds(off, _GCHUNK)], sw_k)
        for h in gv:
            h.wait()
        wv = pltpu.async_copy(rv, sv_hbm.at[pl.ds(off, _GCHUNK)], sw_v)
        wq.wait()
        wk.wait()
        wv.wait()
        return 0

    lax.fori_loop(0, nchunk, chunk, 0)


def _sc_gather(qtab, ktab, vtab, qidx, kidx):
    nrows = qidx.size
    d = qtab.shape[-1]
    mesh = plsc.VectorSubcoreMesh(core_axis_name="c", subcore_axis_name="s")
    out = jax.ShapeDtypeStruct((nrows, d), jnp.float32)
    run = pl.kernel(
        _sc_gather_body,
        out_type=[out, out, out],
        mesh=mesh,
        scratch_types=[
            pltpu.VMEM((_GCHUNK,), jnp.int32),
            pltpu.VMEM((_GCHUNK,), jnp.int32),
            pltpu.VMEM((_GCHUNK, d), jnp.float32),
            pltpu.VMEM((_GCHUNK, d), jnp.float32),
            pltpu.VMEM((_GCHUNK, d), jnp.float32),
        ] + [pltpu.SemaphoreType.DMA] * 6,
        compiler_params=pltpu.CompilerParams(use_tc_tiling_on_sc=False),
    )
    return run(qtab, ktab, vtab, qidx, kidx)


def _sc_scatter_body(bo_hbm, dstidx_hbm, o_hbm, idx_s, rows, sem_r, sem_s):
    nrows = bo_hbm.shape[0]
    gpw = nrows // _NW
    nsuper = gpw // (2 * _SCHUNK)
    npd = _SCHUNK // _IPG
    base = _worker_id() * gpw

    def super_chunk(c, _):
        off0 = base + c * 2 * _SCHUNK
        pltpu.sync_copy(dstidx_hbm.at[pl.ds(off0 // _IPG, 2 * npd)], idx_s)
        for half in range(2):
            off = off0 + half * _SCHUNK
            pltpu.async_copy(bo_hbm.at[pl.ds(off, _SCHUNK)], rows,
                             sem_r).wait()
            ws = [pltpu.async_copy(rows.at[pl.ds(j * _IPG, _IPG)],
                                   o_hbm.at[idx_s.at[half * npd + j]], sem_s)
                  for j in range(npd)]
            for h in ws:
                h.wait()
        return 0

    lax.fori_loop(0, nsuper, super_chunk, 0)


def _sc_scatter(bo2d, dst_rows):
    nrows, w = bo2d.shape
    mesh = plsc.VectorSubcoreMesh(core_axis_name="c", subcore_axis_name="s")
    run = pl.kernel(
        _sc_scatter_body,
        out_type=jax.ShapeDtypeStruct((nrows, w), jnp.float32),
        mesh=mesh,
        scratch_types=[
            pltpu.VMEM((2 * _SCHUNK // _IPG, _IPG), jnp.int32),
            pltpu.VMEM((_SCHUNK, w), jnp.float32),
        ] + [pltpu.SemaphoreType.DMA] * 2,
        compiler_params=pltpu.CompilerParams(use_tc_tiling_on_sc=False),
    )
    return run(bo2d, dst_rows.reshape(-1, _IPG))


_ABLK = 32


def _attn_block_kernel(q_ref, k_ref, v_ref, bo_ref):
    for j in range(_ABLK):
        qp = q_ref[j]
        kp = k_ref[j]
        vp = v_ref[j]
        qe, qo = qp[:, :64], qp[:, 64:]
        ke, ko = kp[:, :64], kp[:, 64:]
        ve, vo = vp[:, :64], vp[:, 64:]

        def half_attn(q):
            se = lax.dot_general(q, ke, (((1,), (1,)), ((), ())),
                                 preferred_element_type=jnp.float32)
            so = lax.dot_general(q, ko, (((1,), (1,)), ((), ())),
                                 preferred_element_type=jnp.float32)
            m = jnp.maximum(jnp.max(se, axis=-1, keepdims=True),
                            jnp.max(so, axis=-1, keepdims=True))
            ee = jnp.exp(se - m)
            eo = jnp.exp(so - m)
            s = (jnp.sum(ee, axis=-1, keepdims=True)
                 + jnp.sum(eo, axis=-1, keepdims=True))
            bo = (lax.dot_general(ee, ve, (((1,), (0,)), ((), ())),
                                  preferred_element_type=jnp.float32)
                  + lax.dot_general(eo, vo, (((1,), (0,)), ((), ())),
                                    preferred_element_type=jnp.float32))
            lse = m + jnp.log(s)
            return jnp.concatenate(
                [bo / s, jnp.broadcast_to(lse, (128, 64))], axis=-1)

        bo_ref[j, 0:128] = half_attn(qe)
        bo_ref[j, 128:256] = half_attn(qo)


def _block_attention(sq, sk, sv, nb):
    g = nb // _ABLK
    spec_in = pl.BlockSpec((_ABLK, 128, 128), lambda i: (i, 0, 0))
    spec_out = pl.BlockSpec((_ABLK, Q_ATTN, 128), lambda i: (i, 0, 0))
    return pl.pallas_call(
        _attn_block_kernel,
        grid=(g,),
        in_specs=[spec_in, spec_in, spec_in],
        out_specs=spec_out,
        out_shape=jax.ShapeDtypeStruct((nb, Q_ATTN, 128), jnp.float32),
    )(sq, sk, sv)


def _combine_kernel(o_ref, out_ref):
    ob = o_ref[:, 0]
    o = ob[:, :, :64]
    l = ob[:, :, 64:65]
    m = jnp.max(l, axis=0, keepdims=True)
    w = jnp.exp(l - m)
    p = w / jnp.sum(w, axis=0, keepdims=True)
    out_ref[0] = jnp.sum(o * p, axis=0)


def _combine(o_pad, bs, n, d):
    h = o_pad.shape[0]
    c = 4096
    return pl.pallas_call(
        _combine_kernel,
        grid=(bs, n // c),
        in_specs=[pl.BlockSpec((h, 1, c, 128), lambda b, i: (0, b, i, 0))],
        out_specs=pl.BlockSpec((1, c, d), lambda b, i: (b, i, 0)),
        out_shape=jax.ShapeDtypeStruct((bs, n, d), jnp.float32),
    )(o_pad)


def _lsh_projections(queries, keys, alpha, beta):
    q_norms = jnp.linalg.norm(queries, axis=-1, keepdims=True)
    k_norms = jnp.linalg.norm(keys, axis=-1, keepdims=True)
    MQ = jnp.max(q_norms, axis=1, keepdims=True)
    MK = jnp.max(k_norms, axis=1, keepdims=True)
    Msq = MQ**2 + MK**2
    ext_q = jnp.sqrt(jnp.maximum(Msq - q_norms**2, 0.0))
    ext_k = jnp.sqrt(jnp.maximum(Msq - k_norms**2, 0.0))
    Q = jnp.concatenate([queries, ext_q, jnp.zeros_like(ext_q)], axis=-1)
    K = jnp.concatenate([keys, jnp.zeros_like(ext_k), ext_k], axis=-1)
    q_proj = jnp.transpose(Q @ alpha + beta, (2, 0, 1))
    k_proj = jnp.transpose(K @ alpha + beta, (2, 0, 1))
    return q_proj, k_proj


_NSPLIT = 2


def kernel(queries, keys, values, alpha, beta):
    bs, n, d = queries.shape
    h = N_HASHES
    q_proj, k_proj = _lsh_projections(queries, keys, alpha, beta)

    qtab = queries.reshape(-1, d)
    ktab = keys.reshape(-1, d)
    vtab = values.reshape(-1, d)

    gbs = bs // _NSPLIT
    outs = []
    for g in range(_NSPLIT):
        bsl = slice(g * gbs, (g + 1) * gbs)
        q_pos = jnp.argsort(q_proj[:, bsl], axis=-1, stable=False)
        k_pos = jnp.argsort(k_proj[:, bsl], axis=-1, stable=False)

        offs = ((jnp.arange(gbs, dtype=jnp.int32) + g * gbs) * n)[None, :, None]
        q_flat = (q_pos + offs).reshape(-1)
        k_flat = (k_pos + offs).reshape(-1)

        sq, sk, sv = _sc_gather(qtab, ktab, vtab, q_flat, k_flat)

        nrows = h * gbs * n
        nb = nrows // Q_ATTN
        bo_pad = _block_attention(sq.reshape(nb, 128, 2 * d),
                                  sk.reshape(nb, 128, 2 * d),
                                  sv.reshape(nb, 128, 2 * d), nb)

        offs2 = (jnp.arange(h * gbs, dtype=jnp.int32) * n)[:, None]
        dst = (q_pos.reshape(-1, n) + offs2).reshape(-1)
        dst_perm = (dst.reshape(nb, 128, 2)
                    .transpose(0, 2, 1).reshape(-1))
        o_pad = _sc_scatter(bo_pad.reshape(nrows, 2 * d), dst_perm)

        outs.append(_combine(o_pad.reshape(h, gbs, n, 2 * d), gbs, n, d))

    return jnp.concatenate(outs, axis=0)


# speedup vs baseline: 1.6603x; 1.6603x over previous
import functools

import jax
import jax.numpy as jnp
from jax import lax
from jax.experimental import pallas as pl
from jax.experimental.pallas import tpu as pltpu
from jax.experimental.pallas import tpu_sc as plsc

N_HASHES = 4
Q_ATTN = 256

_NC = 2
_NS = 16
_NW = _NC * _NS

_IPG = 128
_GCHUNK = 128
_SCHUNK = 512


def _worker_id():
    return lax.axis_index("s") * _NC + lax.axis_index("c")


def _prepack_kernel(q_ref, k_ref, v_ref, qq_ref, kv_ref):
    q = q_ref[...]
    qq_ref[...] = jnp.concatenate([q, q], axis=-1)
    kv_ref[...] = jnp.concatenate([k_ref[...], v_ref[...]], axis=-1)


def _prepack(queries2d, keys2d, values2d):
    nr, d = queries2d.shape
    blk = 4096
    spec_in = pl.BlockSpec((blk, d), lambda i: (i, 0))
    spec_out = pl.BlockSpec((blk, 2 * d), lambda i: (i, 0))
    out = jax.ShapeDtypeStruct((nr, 2 * d), jnp.float32)
    return pl.pallas_call(
        _prepack_kernel,
        grid=(nr // blk,),
        in_specs=[spec_in, spec_in, spec_in],
        out_specs=[spec_out, spec_out],
        out_shape=[out, out],
    )(queries2d, keys2d, values2d)


def _sc_gather_body(qq_hbm, kv_hbm, qidx_hbm, kidx_hbm,
                    sqq_hbm, skv_hbm,
                    idx_q, idx_k, rq0, rkv0, rq1, rkv1,
                    sg_q, sg_k, sw_q, sw_k, sem_i):
    nrows = sqq_hbm.shape[0]
    gpw = nrows // _NW
    npair = gpw // (2 * _GCHUNK)
    base = _worker_id() * gpw

    ci = pltpu.async_copy(qidx_hbm.at[pl.ds(base, gpw)], idx_q, sem_i)
    pltpu.async_copy(kidx_hbm.at[pl.ds(base, gpw)], idx_k, sem_i).wait()
    ci.wait()

    def pair(g, _):
        off0 = base + 2 * g * _GCHUNK
        off1 = off0 + _GCHUNK
        i0 = 2 * g * _GCHUNK
        i1 = i0 + _GCHUNK
        g0q = pltpu.async_copy(qq_hbm.at[idx_q.at[pl.ds(i0, _IPG)]],
                               rq0, sg_q)
        g0k = pltpu.async_copy(kv_hbm.at[idx_k.at[pl.ds(i0, _IPG)]],
                               rkv0, sg_k)
        g1q = pltpu.async_copy(qq_hbm.at[idx_q.at[pl.ds(i1, _IPG)]],
                               rq1, sg_q)
        g1k = pltpu.async_copy(kv_hbm.at[idx_k.at[pl.ds(i1, _IPG)]],
                               rkv1, sg_k)
        g0q.wait()
        w0q = pltpu.async_copy(rq0, sqq_hbm.at[pl.ds(off0, _GCHUNK)], sw_q)
        g0k.wait()
        w0k = pltpu.async_copy(rkv0, skv_hbm.at[pl.ds(off0, _GCHUNK)], sw_k)
        g1q.wait()
        w1q = pltpu.async_copy(rq1, sqq_hbm.at[pl.ds(off1, _GCHUNK)], sw_q)
        g1k.wait()
        w1k = pltpu.async_copy(rkv1, skv_hbm.at[pl.ds(off1, _GCHUNK)], sw_k)
        w0q.wait()
        w0k.wait()
        w1q.wait()
        w1k.wait()
        return 0

    lax.fori_loop(0, npair, pair, 0)


def _sc_gather(qq, kv, qidx, kidx):
    nrows = qidx.size
    w = qq.shape[-1]
    gpw = nrows // _NW
    mesh = plsc.VectorSubcoreMesh(core_axis_name="c", subcore_axis_name="s")
    out = jax.ShapeDtypeStruct((nrows, w), jnp.float32)
    run = pl.kernel(
        _sc_gather_body,
        out_type=[out, out],
        mesh=mesh,
        scratch_types=[
            pltpu.VMEM((gpw,), jnp.int32),
            pltpu.VMEM((gpw,), jnp.int32),
            pltpu.VMEM((_GCHUNK, w), jnp.float32),
            pltpu.VMEM((_GCHUNK, w), jnp.float32),
            pltpu.VMEM((_GCHUNK, w), jnp.float32),
            pltpu.VMEM((_GCHUNK, w), jnp.float32),
        ] + [pltpu.SemaphoreType.DMA] * 5,
        compiler_params=pltpu.CompilerParams(use_tc_tiling_on_sc=False),
    )
    return run(qq, kv, qidx, kidx)


def _sc_scatter_body(bo_hbm, dstidx_hbm, o_hbm, idx_s, rows, sem_r, sem_s):
    nrows = bo_hbm.shape[0]
    gpw = nrows // _NW
    nsuper = gpw // (2 * _SCHUNK)
    npd = _SCHUNK // _IPG
    base = _worker_id() * gpw

    def super_chunk(c, _):
        off0 = base + c * 2 * _SCHUNK
        pltpu.sync_copy(dstidx_hbm.at[pl.ds(off0 // _IPG, 2 * npd)], idx_s)
        for half in range(2):
            off = off0 + half * _SCHUNK
            pltpu.async_copy(bo_hbm.at[pl.ds(off, _SCHUNK)], rows,
                             sem_r).wait()
            ws = [pltpu.async_copy(rows.at[pl.ds(j * _IPG, _IPG)],
                                   o_hbm.at[idx_s.at[half * npd + j]], sem_s)
                  for j in range(npd)]
            for h in ws:
                h.wait()
        return 0

    lax.fori_loop(0, nsuper, super_chunk, 0)


def _sc_scatter(bo2d, dst_rows):
    nrows, w = bo2d.shape
    mesh = plsc.VectorSubcoreMesh(core_axis_name="c", subcore_axis_name="s")
    run = pl.kernel(
        _sc_scatter_body,
        out_type=jax.ShapeDtypeStruct((nrows, w), jnp.float32),
        mesh=mesh,
        scratch_types=[
            pltpu.VMEM((2 * _SCHUNK // _IPG, _IPG), jnp.int32),
            pltpu.VMEM((_SCHUNK, w), jnp.float32),
        ] + [pltpu.SemaphoreType.DMA] * 2,
        compiler_params=pltpu.CompilerParams(use_tc_tiling_on_sc=False),
    )
    return run(bo2d, dst_rows.reshape(-1, _IPG))


_ABLK = 32


def _attn_block_kernel(qq_ref, kv_ref, bo_ref):
    for j in range(_ABLK):
        q = qq_ref[j, :, :64]
        k = kv_ref[j, :, :64]
        v = kv_ref[j, :, 64:]
        inner = lax.dot_general(q, k, (((1,), (1,)), ((), ())),
                                preferred_element_type=jnp.float32)
        m = jnp.max(inner, axis=-1, keepdims=True)
        e = jnp.exp(inner - m)
        s = jnp.sum(e, axis=-1, keepdims=True)
        bo = lax.dot_general(e, v, (((1,), (0,)), ((), ())),
                             preferred_element_type=jnp.float32)
        lse = m + jnp.log(s)
        bo_ref[j] = jnp.concatenate(
            [bo / s, jnp.broadcast_to(lse, (Q_ATTN, 64))], axis=-1)


def _block_attention(sqq, skv):
    nb = sqq.shape[0]
    g = nb // _ABLK
    spec = pl.BlockSpec((_ABLK, Q_ATTN, 128), lambda i: (i, 0, 0))
    return pl.pallas_call(
        _attn_block_kernel,
        grid=(g,),
        in_specs=[spec, spec],
        out_specs=spec,
        out_shape=jax.ShapeDtypeStruct((nb, Q_ATTN, 128), jnp.float32),
    )(sqq, skv)


def _combine_kernel(o_ref, out_ref):
    ob = o_ref[:, 0]
    o = ob[:, :, :64]
    l = ob[:, :, 64:65]
    m = jnp.max(l, axis=0, keepdims=True)
    w = jnp.exp(l - m)
    p = w / jnp.sum(w, axis=0, keepdims=True)
    out_ref[0] = jnp.sum(o * p, axis=0)


def _combine(o_pad, bs, n, d):
    h = o_pad.shape[0]
    c = 4096
    return pl.pallas_call(
        _combine_kernel,
        grid=(bs, n // c),
        in_specs=[pl.BlockSpec((h, 1, c, 128), lambda b, i: (0, b, i, 0))],
        out_specs=pl.BlockSpec((1, c, d), lambda b, i: (b, i, 0)),
        out_shape=jax.ShapeDtypeStruct((bs, n, d), jnp.float32),
    )(o_pad)


def _lsh_positions(queries, keys, alpha, beta):
    q_norms = jnp.linalg.norm(queries, axis=-1, keepdims=True)
    k_norms = jnp.linalg.norm(keys, axis=-1, keepdims=True)
    MQ = jnp.max(q_norms, axis=1, keepdims=True)
    MK = jnp.max(k_norms, axis=1, keepdims=True)
    Msq = MQ**2 + MK**2
    ext_q = jnp.sqrt(jnp.maximum(Msq - q_norms**2, 0.0))
    ext_k = jnp.sqrt(jnp.maximum(Msq - k_norms**2, 0.0))
    Q = jnp.concatenate([queries, ext_q, jnp.zeros_like(ext_q)], axis=-1)
    K = jnp.concatenate([keys, jnp.zeros_like(ext_k), ext_k], axis=-1)
    q_proj = jnp.transpose(Q @ alpha + beta, (2, 0, 1))
    k_proj = jnp.transpose(K @ alpha + beta, (2, 0, 1))
    return jnp.argsort(q_proj, axis=-1), jnp.argsort(k_proj, axis=-1)


def _lsh_projections(queries, keys, alpha, beta):
    q_norms = jnp.linalg.norm(queries, axis=-1, keepdims=True)
    k_norms = jnp.linalg.norm(keys, axis=-1, keepdims=True)
    MQ = jnp.max(q_norms, axis=1, keepdims=True)
    MK = jnp.max(k_norms, axis=1, keepdims=True)
    Msq = MQ**2 + MK**2
    ext_q = jnp.sqrt(jnp.maximum(Msq - q_norms**2, 0.0))
    ext_k = jnp.sqrt(jnp.maximum(Msq - k_norms**2, 0.0))
    Q = jnp.concatenate([queries, ext_q, jnp.zeros_like(ext_q)], axis=-1)
    K = jnp.concatenate([keys, jnp.zeros_like(ext_k), ext_k], axis=-1)
    q_proj = jnp.transpose(Q @ alpha + beta, (2, 0, 1))
    k_proj = jnp.transpose(K @ alpha + beta, (2, 0, 1))
    return q_proj, k_proj


_NSPLIT = 2


def kernel(queries, keys, values, alpha, beta):
    bs, n, d = queries.shape
    h = N_HASHES
    q_proj, k_proj = _lsh_projections(queries, keys, alpha, beta)

    qq, kv = _prepack(queries.reshape(-1, d), keys.reshape(-1, d),
                      values.reshape(-1, d))

    gbs = bs // _NSPLIT
    outs = []
    for g in range(_NSPLIT):
        bsl = slice(g * gbs, (g + 1) * gbs)
        q_pos = jnp.argsort(q_proj[:, bsl], axis=-1, stable=False)
        k_pos = jnp.argsort(k_proj[:, bsl], axis=-1, stable=False)

        offs = ((jnp.arange(gbs, dtype=jnp.int32) + g * gbs) * n)[None, :, None]
        q_flat = (q_pos + offs).reshape(-1)
        k_flat = (k_pos + offs).reshape(-1)

        sqq, skv = _sc_gather(qq, kv, q_flat, k_flat)

        bo_pad = _block_attention(sqq.reshape(-1, Q_ATTN, 2 * d),
                                  skv.reshape(-1, Q_ATTN, 2 * d))

        offs2 = (jnp.arange(h * gbs, dtype=jnp.int32) * n)[:, None]
        dst_rows = (q_pos.reshape(-1, n) + offs2).reshape(-1)
        o_pad = _sc_scatter(bo_pad.reshape(-1, 2 * d), dst_rows)

        outs.append(_combine(o_pad.reshape(h, gbs, n, 2 * d), gbs, n, d))

    return jnp.concatenate(outs, axis=0)
